# Initial kernel scaffold; baseline (speedup 1.0000x reference)
import jax
import jax.numpy as jnp
from jax import lax
from jax.experimental import pallas as pl
from jax.experimental.pallas import tpu as pltpu
from jax.experimental.pallas import tpu_sc as plsc

N_ = 10000
NP_ = 10240
R_ = 2 * NP_
E_ = 320000
SUB = 16
CH = 128
EPW = ((E_ // SUB) + CH - 1) // CH * CH
NCH = EPW // CH
EPAD = SUB * EPW
RPW = NP_ // SUB
H_ = 128
G_ = 256
EPS = 1e-5
BR = 2048
GRID = R_ // BR

_mesh = plsc.VectorSubcoreMesh(core_axis_name="c", subcore_axis_name="s")


def _deg_body(di_hbm, ones_hbm, zeros1_hbm, deg_out, di_buf, ones_v, deg_sh):
    c = lax.axis_index("c")
    s = lax.axis_index("s")
    pltpu.sync_copy(di_hbm.at[c, s], di_buf)
    pltpu.sync_copy(ones_hbm, ones_v)

    @pl.when(s == 0)
    def _():
        pltpu.sync_copy(zeros1_hbm, deg_sh)

    plsc.subcore_barrier()

    def body(j, carry):
        pltpu.sync_copy(ones_v, deg_sh.at[di_buf.at[j]], add=True)
        return carry

    lax.fori_loop(0, NCH, body, 0)
    plsc.subcore_barrier()
    pltpu.sync_copy(deg_sh.at[pl.ds(s * RPW, RPW)],
                    deg_out.at[c, pl.ds(s * RPW, RPW)])


_deg_call = pl.kernel(
    _deg_body,
    out_type=jax.ShapeDtypeStruct((2, NP_), jnp.float32),
    mesh=_mesh,
    scratch_types=[
        pltpu.VMEM((NCH, CH), jnp.int32),
        pltpu.VMEM((CH,), jnp.float32),
        pltpu.VMEM_SHARED((NP_,), jnp.float32),
    ],
)


def _scat_body(p_hbm, si_hbm, di_hbm, zerosf_hbm, s_out,
               si_buf, di_buf, rows_a, rows_b, acc_sh, sem_a, sem_b):
    c = lax.axis_index("c")
    s = lax.axis_index("s")
    pltpu.sync_copy(si_hbm.at[c, s], si_buf)
    pltpu.sync_copy(di_hbm.at[c, s], di_buf)
    pltpu.sync_copy(zerosf_hbm.at[pl.ds(s * RPW, RPW)],
                    acc_sh.at[pl.ds(s * RPW, RPW)])
    plsc.subcore_barrier()

    pltpu.async_copy(p_hbm.at[si_buf.at[0]], rows_a, sem_a)

    def body(j, carry):
        even = lax.rem(j, 2) == 0

        @pl.when(j + 1 < NCH)
        def _():
            @pl.when(even)
            def _():
                pltpu.async_copy(p_hbm.at[si_buf.at[j + 1]], rows_b, sem_b)

            @pl.when(jnp.logical_not(even))
            def _():
                pltpu.async_copy(p_hbm.at[si_buf.at[j + 1]], rows_a, sem_a)

        @pl.when(even)
        def _():
            pltpu.make_async_copy(p_hbm.at[si_buf.at[0]], rows_a, sem_a).wait()
            pltpu.sync_copy(rows_a, acc_sh.at[di_buf.at[j]], add=True)

        @pl.when(jnp.logical_not(even))
        def _():
            pltpu.make_async_copy(p_hbm.at[si_buf.at[0]], rows_b, sem_b).wait()
            pltpu.sync_copy(rows_b, acc_sh.at[di_buf.at[j]], add=True)

        return carry

    lax.fori_loop(0, NCH, body, 0)
    plsc.subcore_barrier()
    pltpu.sync_copy(acc_sh.at[pl.ds(s * RPW, RPW)],
                    s_out.at[c, pl.ds(s * RPW, RPW)])


_scat_call = pl.kernel(
    _scat_body,
    out_type=jax.ShapeDtypeStruct((2, NP_, H_), jnp.float32),
    mesh=_mesh,
    scratch_types=[
        pltpu.VMEM((NCH, CH), jnp.int32),
        pltpu.VMEM((NCH, CH), jnp.int32),
        pltpu.VMEM((CH, H_), jnp.float32),
        pltpu.VMEM((CH, H_), jnp.float32),
        pltpu.VMEM_SHARED((NP_, H_), jnp.float32),
        pltpu.SemaphoreType.DMA,
        pltpu.SemaphoreType.DMA,
    ],
)


def _first_body(x_ref, deg_ref, mask_ref, w_ref, p_ref):
    dis = lax.rsqrt(deg_ref[...] + 1.0)
    q = jnp.dot(x_ref[...], w_ref[...], preferred_element_type=jnp.float32)
    p_ref[...] = q * dis * mask_ref[...]


def _layer_body(s_ref, p_ref, deg_ref, mask_ref, w_ref, b_ref, g_ref, be_ref,
                out_ref):
    dis = lax.rsqrt(deg_ref[...] + 1.0)
    z = dis * (s_ref[...] + p_ref[...]) + b_ref[...]
    z = z * (g_ref[...] * lax.rsqrt(1.0 + EPS)) + be_ref[...]
    f = jnp.maximum(z, 0.0)
    q = jnp.dot(f, w_ref[...], preferred_element_type=jnp.float32)
    out_ref[...] = q * dis * mask_ref[...]


def _head_body(s_ref, p_ref, deg_ref, batch_ref,
               b3_ref, g3_ref, be3_ref,
               wc1_ref, bc1_ref, gc1_ref, bec1_ref,
               wc2_ref, bc2_ref, gc2_ref, bec2_ref,
               wc3_ref, bc3_ref,
               out_ref, pool_s, cnt_s):
    j = pl.program_id(0)

    @pl.when(j == 0)
    def _():
        pool_s[...] = jnp.zeros_like(pool_s)
        cnt_s[...] = jnp.zeros_like(cnt_s)

    dis = lax.rsqrt(deg_ref[...] + 1.0)
    z = dis * (s_ref[...] + p_ref[...]) + b3_ref[...]
    z = z * (g3_ref[...] * lax.rsqrt(1.0 + EPS)) + be3_ref[...]
    h = jnp.maximum(z, 0.0)

    gid = lax.broadcasted_iota(jnp.int32, (BR, 2 * G_), 1)
    m = (batch_ref[...] == gid).astype(jnp.float32)
    dn = (((0,), (0,)), ((), ()))
    pool_s[...] += lax.dot_general(m, h, dn, preferred_element_type=jnp.float32)
    cnt_s[...] += lax.dot_general(m, jnp.ones_like(h), dn,
                                  preferred_element_type=jnp.float32)

    @pl.when(j == GRID - 1)
    def _():
        emb = pool_s[...] / jnp.maximum(cnt_s[...], 1.0)
        comb = jnp.concatenate([emb[0:G_, :], emb[G_:2 * G_, :]], axis=1)
        z1 = jnp.dot(comb, wc1_ref[...], preferred_element_type=jnp.float32)
        z1 = z1 + bc1_ref[...]
        z1 = z1 * (gc1_ref[...] * lax.rsqrt(1.0 + EPS)) + bec1_ref[...]
        z1 = jnp.maximum(z1, 0.0)
        z2 = jnp.dot(z1, wc2_ref[...], preferred_element_type=jnp.float32)
        z2 = z2 + bc2_ref[...]
        z2 = z2 * (gc2_ref[...] * lax.rsqrt(1.0 + EPS)) + bec2_ref[...]
        z2 = jnp.maximum(z2, 0.0)
        z3 = jnp.dot(z2, wc3_ref[...], preferred_element_type=jnp.float32)
        out_ref[...] = z3 + bc3_ref[...]


def _row_spec(width):
    return pl.BlockSpec((BR, width), lambda j: (j, 0))


def _full_spec(shape):
    return pl.BlockSpec(shape, lambda j: tuple(0 for _ in shape))


def _first_call(xp, deg, mask, w1p):
    return pl.pallas_call(
        _first_body,
        grid=(GRID,),
        in_specs=[_row_spec(8), _row_spec(1), _row_spec(1), _full_spec((8, H_))],
        out_specs=_row_spec(H_),
        out_shape=jax.ShapeDtypeStruct((R_, H_), jnp.float32),
    )(xp, deg, mask, w1p)


def _layer_call(s, p, deg, mask, w, b, g, be):
    return pl.pallas_call(
        _layer_body,
        grid=(GRID,),
        in_specs=[_row_spec(H_), _row_spec(H_), _row_spec(1), _row_spec(1),
                  _full_spec((H_, H_)), _full_spec((1, H_)),
                  _full_spec((1, H_)), _full_spec((1, H_))],
        out_specs=_row_spec(H_),
        out_shape=jax.ShapeDtypeStruct((R_, H_), jnp.float32),
    )(s, p, deg, mask, w, b, g, be)


def _head_call(s, p, deg, batchp, b3, g3, be3,
               wc1, bc1, gc1, bec1, wc2, bc2, gc2, bec2, wc3p, bc3p):
    return pl.pallas_call(
        _head_body,
        grid=(GRID,),
        in_specs=[_row_spec(H_), _row_spec(H_), _row_spec(1), _row_spec(1),
                  _full_spec((1, H_)), _full_spec((1, H_)), _full_spec((1, H_)),
                  _full_spec((2 * H_, H_)), _full_spec((1, H_)),
                  _full_spec((1, H_)), _full_spec((1, H_)),
                  _full_spec((H_, H_ // 2)), _full_spec((1, H_ // 2)),
                  _full_spec((1, H_ // 2)), _full_spec((1, H_ // 2)),
                  _full_spec((H_ // 2, H_)), _full_spec((1, H_))],
        out_specs=_full_spec((G_, H_)),
        out_shape=jax.ShapeDtypeStruct((G_, H_), jnp.float32),
        scratch_shapes=[pltpu.VMEM((2 * G_, H_), jnp.float32),
                        pltpu.VMEM((2 * G_, H_), jnp.float32)],
    )(s, p, deg, batchp, b3, g3, be3,
      wc1, bc1, gc1, bec1, wc2, bc2, gc2, bec2, wc3p, bc3p)


def _pad_edges(src, dst, coff):
    npad = EPAD - E_
    si = jnp.concatenate([src + coff, jnp.full((npad,), coff + N_, jnp.int32)])
    di = jnp.concatenate([dst, jnp.full((npad,), N_, jnp.int32)])
    return si.reshape(SUB, NCH, CH), di.reshape(SUB, NCH, CH)


def kernel(x1, edge_index1, edge_attr1, batch1, x2, edge_index2, edge_attr2,
           batch2, W1, b1, g1, be1, W2, b2, g2, be2, W3, b3, g3, be3,
           Wc1, bc1, gc1, bec1, Wc2, bc2, gc2, bec2, Wc3, bc3):
    f32 = jnp.float32
    xp = jnp.zeros((R_, 8), f32)
    xp = xp.at[0:N_, 0:6].set(x1).at[NP_:NP_ + N_, 0:6].set(x2)
    mask = jnp.zeros((R_, 1), f32)
    mask = mask.at[0:N_].set(1.0).at[NP_:NP_ + N_].set(1.0)
    batchp = jnp.full((R_, 1), 2 * G_, jnp.int32)
    batchp = batchp.at[0:N_, 0].set(batch1).at[NP_:NP_ + N_, 0].set(batch2 + G_)

    si1, di1 = _pad_edges(edge_index1[0], edge_index1[1], 0)
    si2, di2 = _pad_edges(edge_index2[0], edge_index2[1], NP_)
    si = jnp.stack([si1, si2])
    di = jnp.stack([di1, di2])

    ones_h = jnp.ones((CH,), f32)
    zeros1 = jnp.zeros((NP_,), f32)
    zerosf = jnp.zeros((NP_, H_), f32)

    w1p = jnp.zeros((8, H_), f32).at[0:6].set(W1)
    wc3p = jnp.zeros((H_ // 2, H_), f32).at[:, 0:1].set(Wc3)
    bc3p = jnp.zeros((1, H_), f32).at[0, 0].set(bc3[0])
    row = lambda v: v.reshape(1, -1)

    deg = _deg_call(di, ones_h, zeros1).reshape(R_, 1)

    p1 = _first_call(xp, deg, mask, w1p)
    s1 = _scat_call(p1, si, di, zerosf).reshape(R_, H_)
    p2 = _layer_call(s1, p1, deg, mask, W2, row(b1), row(g1), row(be1))
    s2 = _scat_call(p2, si, di, zerosf).reshape(R_, H_)
    p3 = _layer_call(s2, p2, deg, mask, W3, row(b2), row(g2), row(be2))
    s3 = _scat_call(p3, si, di, zerosf).reshape(R_, H_)

    out = _head_call(s3, p3, deg, batchp,
                     row(b3), row(g3), row(be3),
                     Wc1, row(bc1), row(gc1), row(bec1),
                     Wc2, row(bc2), row(gc2), row(bec2),
                     wc3p, bc3p)
    return out[:, 0:1]


# speedup vs baseline: 17.0528x; 17.0528x over previous
import jax
import jax.numpy as jnp
from jax import lax
from jax.experimental import pallas as pl
from jax.experimental.pallas import tpu as pltpu
from jax.experimental.pallas import tpu_sc as plsc

N_ = 10000
NP_ = 10240
R_ = 2 * NP_
E_ = 320000
SUB = 16
CH = 128
EPW = ((E_ // SUB) + CH - 1) // CH * CH
NCH = EPW // CH
EPAD = SUB * EPW
RPW = NP_ // SUB
H_ = 128
G_ = 256
EPS = 1e-5
BR = 2048
GRID = R_ // BR

_mesh = plsc.VectorSubcoreMesh(core_axis_name="c", subcore_axis_name="s")


def _deg_body(sidi_hbm, ones_hbm, zeros1_hbm, deg_out, idx_buf, ones_v, deg_sh):
    c = lax.axis_index("c")
    s = lax.axis_index("s")
    pltpu.sync_copy(sidi_hbm.at[c, s], idx_buf)
    pltpu.sync_copy(ones_hbm, ones_v)

    @pl.when(s == 0)
    def _():
        pltpu.sync_copy(zeros1_hbm, deg_sh)

    plsc.subcore_barrier()

    def body(j, carry):
        pltpu.sync_copy(ones_v, deg_sh.at[idx_buf.at[j, 1]], add=True)
        return carry

    lax.fori_loop(0, NCH, body, 0)
    plsc.subcore_barrier()
    pltpu.sync_copy(deg_sh.at[pl.ds(s * RPW, RPW)],
                    deg_out.at[c, pl.ds(s * RPW, RPW)])


_deg_call = pl.kernel(
    _deg_body,
    out_type=jax.ShapeDtypeStruct((2, NP_), jnp.float32),
    mesh=_mesh,
    scratch_types=[
        pltpu.VMEM((NCH, 2, CH), jnp.int32),
        pltpu.VMEM((CH,), jnp.float32),
        pltpu.VMEM_SHARED((NP_,), jnp.float32),
    ],
)


def _scat_body(p_hbm, sidi_hbm, zerosf_hbm, s_out,
               ib_a, ib_b, rows_a, rows_b, acc_sh, sem_a, sem_b, sem_i):
    c = lax.axis_index("c")
    s = lax.axis_index("s")
    pltpu.sync_copy(zerosf_hbm.at[pl.ds(s * RPW, RPW)],
                    acc_sh.at[pl.ds(s * RPW, RPW)])
    plsc.subcore_barrier()

    pltpu.sync_copy(sidi_hbm.at[c, s, 0], ib_a)
    pltpu.async_copy(p_hbm.at[ib_a.at[0]], rows_a, sem_a)
    pltpu.async_copy(sidi_hbm.at[c, s, 1], ib_b, sem_i)

    def body(j, carry):
        even = lax.rem(j, 2) == 0
        odd = jnp.logical_not(even)

        @pl.when(even)
        def _():
            pltpu.make_async_copy(p_hbm.at[ib_a.at[0]], rows_a, sem_a).wait()

        @pl.when(odd)
        def _():
            pltpu.make_async_copy(p_hbm.at[ib_a.at[0]], rows_b, sem_b).wait()

        @pl.when(j + 1 < NCH)
        def _():
            pltpu.make_async_copy(sidi_hbm.at[c, s, 0], ib_a, sem_i).wait()

            @pl.when(even)
            def _():
                pltpu.async_copy(p_hbm.at[ib_b.at[0]], rows_b, sem_b)

            @pl.when(odd)
            def _():
                pltpu.async_copy(p_hbm.at[ib_a.at[0]], rows_a, sem_a)

        @pl.when(even)
        def _():
            pltpu.sync_copy(rows_a, acc_sh.at[ib_a.at[1]], add=True)

        @pl.when(odd)
        def _():
            pltpu.sync_copy(rows_b, acc_sh.at[ib_b.at[1]], add=True)

        @pl.when(j + 2 < NCH)
        def _():
            @pl.when(even)
            def _():
                pltpu.async_copy(sidi_hbm.at[c, s, j + 2], ib_a, sem_i)

            @pl.when(odd)
            def _():
                pltpu.async_copy(sidi_hbm.at[c, s, j + 2], ib_b, sem_i)

        return carry

    lax.fori_loop(0, NCH, body, 0)
    plsc.subcore_barrier()
    pltpu.sync_copy(acc_sh.at[pl.ds(s * RPW, RPW)],
                    s_out.at[c, pl.ds(s * RPW, RPW)])


_scat_call = pl.kernel(
    _scat_body,
    out_type=jax.ShapeDtypeStruct((2, NP_, H_), jnp.float32),
    mesh=_mesh,
    scratch_types=[
        pltpu.VMEM((2, CH), jnp.int32),
        pltpu.VMEM((2, CH), jnp.int32),
        pltpu.VMEM((CH, H_), jnp.float32),
        pltpu.VMEM((CH, H_), jnp.float32),
        pltpu.VMEM_SHARED((NP_, H_), jnp.float32),
        pltpu.SemaphoreType.DMA,
        pltpu.SemaphoreType.DMA,
        pltpu.SemaphoreType.DMA,
    ],
)


def _first_body(x_ref, deg_ref, mask_ref, w_ref, p_ref):
    dis = lax.rsqrt(deg_ref[...] + 1.0)
    q = jnp.dot(x_ref[...], w_ref[...], preferred_element_type=jnp.float32)
    p_ref[...] = q * dis * mask_ref[...]


def _layer_body(s_ref, p_ref, deg_ref, mask_ref, w_ref, b_ref, g_ref, be_ref,
                out_ref):
    dis = lax.rsqrt(deg_ref[...] + 1.0)
    z = dis * (s_ref[...] + p_ref[...]) + b_ref[...]
    z = z * (g_ref[...] * lax.rsqrt(1.0 + EPS)) + be_ref[...]
    f = jnp.maximum(z, 0.0)
    q = jnp.dot(f, w_ref[...], preferred_element_type=jnp.float32)
    out_ref[...] = q * dis * mask_ref[...]


def _head_body(s_ref, p_ref, deg_ref, batch_ref,
               b3_ref, g3_ref, be3_ref,
               wc1_ref, bc1_ref, gc1_ref, bec1_ref,
               wc2_ref, bc2_ref, gc2_ref, bec2_ref,
               wc3_ref, bc3_ref,
               out_ref, pool_s, cnt_s):
    j = pl.program_id(0)

    @pl.when(j == 0)
    def _():
        pool_s[...] = jnp.zeros_like(pool_s)
        cnt_s[...] = jnp.zeros_like(cnt_s)

    dis = lax.rsqrt(deg_ref[...] + 1.0)
    z = dis * (s_ref[...] + p_ref[...]) + b3_ref[...]
    z = z * (g3_ref[...] * lax.rsqrt(1.0 + EPS)) + be3_ref[...]
    h = jnp.maximum(z, 0.0)

    gid = lax.broadcasted_iota(jnp.int32, (BR, 2 * G_), 1)
    m = (batch_ref[...] == gid).astype(jnp.float32)
    dn = (((0,), (0,)), ((), ()))
    pool_s[...] += lax.dot_general(m, h, dn, preferred_element_type=jnp.float32)
    cnt_s[...] += lax.dot_general(m, jnp.ones_like(h), dn,
                                  preferred_element_type=jnp.float32)

    @pl.when(j == GRID - 1)
    def _():
        emb = pool_s[...] / jnp.maximum(cnt_s[...], 1.0)
        comb = jnp.concatenate([emb[0:G_, :], emb[G_:2 * G_, :]], axis=1)
        z1 = jnp.dot(comb, wc1_ref[...], preferred_element_type=jnp.float32)
        z1 = z1 + bc1_ref[...]
        z1 = z1 * (gc1_ref[...] * lax.rsqrt(1.0 + EPS)) + bec1_ref[...]
        z1 = jnp.maximum(z1, 0.0)
        z2 = jnp.dot(z1, wc2_ref[...], preferred_element_type=jnp.float32)
        z2 = z2 + bc2_ref[...]
        z2 = z2 * (gc2_ref[...] * lax.rsqrt(1.0 + EPS)) + bec2_ref[...]
        z2 = jnp.maximum(z2, 0.0)
        z3 = jnp.dot(z2, wc3_ref[...], preferred_element_type=jnp.float32)
        out_ref[...] = z3 + bc3_ref[...]


def _row_spec(width):
    return pl.BlockSpec((BR, width), lambda j: (j, 0))


def _full_spec(shape):
    return pl.BlockSpec(shape, lambda j: tuple(0 for _ in shape))


def _first_call(xp, deg, mask, w1p):
    return pl.pallas_call(
        _first_body,
        grid=(GRID,),
        in_specs=[_row_spec(8), _row_spec(1), _row_spec(1), _full_spec((8, H_))],
        out_specs=_row_spec(H_),
        out_shape=jax.ShapeDtypeStruct((R_, H_), jnp.float32),
    )(xp, deg, mask, w1p)


def _layer_call(s, p, deg, mask, w, b, g, be):
    return pl.pallas_call(
        _layer_body,
        grid=(GRID,),
        in_specs=[_row_spec(H_), _row_spec(H_), _row_spec(1), _row_spec(1),
                  _full_spec((H_, H_)), _full_spec((1, H_)),
                  _full_spec((1, H_)), _full_spec((1, H_))],
        out_specs=_row_spec(H_),
        out_shape=jax.ShapeDtypeStruct((R_, H_), jnp.float32),
    )(s, p, deg, mask, w, b, g, be)


def _head_call(s, p, deg, batchp, b3, g3, be3,
               wc1, bc1, gc1, bec1, wc2, bc2, gc2, bec2, wc3p, bc3p):
    return pl.pallas_call(
        _head_body,
        grid=(GRID,),
        in_specs=[_row_spec(H_), _row_spec(H_), _row_spec(1), _row_spec(1),
                  _full_spec((1, H_)), _full_spec((1, H_)), _full_spec((1, H_)),
                  _full_spec((2 * H_, H_)), _full_spec((1, H_)),
                  _full_spec((1, H_)), _full_spec((1, H_)),
                  _full_spec((H_, H_ // 2)), _full_spec((1, H_ // 2)),
                  _full_spec((1, H_ // 2)), _full_spec((1, H_ // 2)),
                  _full_spec((H_ // 2, H_)), _full_spec((1, H_))],
        out_specs=_full_spec((G_, H_)),
        out_shape=jax.ShapeDtypeStruct((G_, H_), jnp.float32),
        scratch_shapes=[pltpu.VMEM((2 * G_, H_), jnp.float32),
                        pltpu.VMEM((2 * G_, H_), jnp.float32)],
    )(s, p, deg, batchp, b3, g3, be3,
      wc1, bc1, gc1, bec1, wc2, bc2, gc2, bec2, wc3p, bc3p)


def _pad_edges(src, dst, coff):
    npad = EPAD - E_
    si = jnp.concatenate([src + coff, jnp.full((npad,), coff + N_, jnp.int32)])
    di = jnp.concatenate([dst, jnp.full((npad,), N_, jnp.int32)])
    return jnp.stack([si.reshape(SUB, NCH, CH), di.reshape(SUB, NCH, CH)],
                     axis=2)


def kernel(x1, edge_index1, edge_attr1, batch1, x2, edge_index2, edge_attr2,
           batch2, W1, b1, g1, be1, W2, b2, g2, be2, W3, b3, g3, be3,
           Wc1, bc1, gc1, bec1, Wc2, bc2, gc2, bec2, Wc3, bc3):
    f32 = jnp.float32
    xp = jnp.zeros((R_, 8), f32)
    xp = xp.at[0:N_, 0:6].set(x1).at[NP_:NP_ + N_, 0:6].set(x2)
    mask = jnp.zeros((R_, 1), f32)
    mask = mask.at[0:N_].set(1.0).at[NP_:NP_ + N_].set(1.0)
    batchp = jnp.full((R_, 1), 2 * G_, jnp.int32)
    batchp = batchp.at[0:N_, 0].set(batch1).at[NP_:NP_ + N_, 0].set(batch2 + G_)

    sidi = jnp.stack([_pad_edges(edge_index1[0], edge_index1[1], 0),
                      _pad_edges(edge_index2[0], edge_index2[1], NP_)])

    ones_h = jnp.ones((CH,), f32)
    zeros1 = jnp.zeros((NP_,), f32)
    zerosf = jnp.zeros((NP_, H_), f32)

    w1p = jnp.zeros((8, H_), f32).at[0:6].set(W1)
    wc3p = jnp.zeros((H_ // 2, H_), f32).at[:, 0:1].set(Wc3)
    bc3p = jnp.zeros((1, H_), f32).at[0, 0].set(bc3[0])
    row = lambda v: v.reshape(1, -1)

    deg = _deg_call(sidi, ones_h, zeros1).reshape(R_, 1)

    p1 = _first_call(xp, deg, mask, w1p)
    s1 = _scat_call(p1, sidi, zerosf).reshape(R_, H_)
    p2 = _layer_call(s1, p1, deg, mask, W2, row(b1), row(g1), row(be1))
    s2 = _scat_call(p2, sidi, zerosf).reshape(R_, H_)
    p3 = _layer_call(s2, p2, deg, mask, W3, row(b2), row(g2), row(be2))
    s3 = _scat_call(p3, sidi, zerosf).reshape(R_, H_)

    out = _head_call(s3, p3, deg, batchp,
                     row(b3), row(g3), row(be3),
                     Wc1, row(bc1), row(gc1), row(bec1),
                     Wc2, row(bc2), row(gc2), row(bec2),
                     wc3p, bc3p)
    return out[:, 0:1]
